# baseline (speedup 1.0000x reference)
import functools

import jax
import jax.numpy as jnp
from jax import lax
from jax.experimental import pallas as pl
from jax.experimental.pallas import tpu as pltpu
from jax.experimental.pallas import tpu_sc as plsc

_N = 8388608
_W = 672
_NUM_CASES = 512


_LANES = 128
_ROWS = _N // _LANES
_BLK_R = 8192
_GRID = _ROWS // _BLK_R


def _tc_body(p_ref, t_ref, out_ref):
    k = pl.program_id(0)

    @pl.when(k == 0)
    def _init():
        out_ref[...] = jnp.zeros((1, 1), jnp.float32)

    d = p_ref[...] - t_ref[...]
    out_ref[...] = out_ref[...] + jnp.sum(d * d).reshape(1, 1)


def _tc_sum_sq(pred, true):
    p2 = pred.reshape(_ROWS, _LANES)
    t2 = true.reshape(_ROWS, _LANES)
    spec = pl.BlockSpec((_BLK_R, _LANES), lambda k: (k, 0))
    total = pl.pallas_call(
        _tc_body,
        grid=(_GRID,),
        in_specs=[spec, spec],
        out_specs=pl.BlockSpec((1, 1), lambda k: (0, 0)),
        out_shape=jax.ShapeDtypeStruct((1, 1), jnp.float32),
    )(p2, t2)
    return total[0, 0]


_NW = 32
_CPW = _NUM_CASES // _NW
_NVEC = _W // 16
_SEARCH_ROUNDS = 24
_CHUNKS = [(0, 128), (128, 128), (256, 128), (384, 128), (512, 128),
           (640, 32)]


def _sc_boundary_kernel():
    mesh = plsc.VectorSubcoreMesh(core_axis_name="c", subcore_axis_name="s")

    @functools.partial(
        pl.kernel,
        mesh=mesh,
        out_type=jax.ShapeDtypeStruct((_NW, 16), jnp.float32),
        scratch_types=[
            pltpu.VMEM((32,), jnp.int32),
            pltpu.VMEM((32,), jnp.int32),
            pltpu.VMEM((_W,), jnp.int32),
            pltpu.VMEM((_W,), jnp.float32),
            pltpu.VMEM((_W,), jnp.float32),
            pltpu.VMEM((16,), jnp.float32),
            pltpu.SemaphoreType.DMA,
        ],
    )
    def sc_kernel(batch_hbm, pred_hbm, true_hbm, out_hbm,
                  idx_v, val_v, widx, pbuf, tbuf, acc_v, sem):
        cid = lax.axis_index("c")
        sid = lax.axis_index("s")
        wid = sid * 2 + cid

        lane = jnp.arange(16, dtype=jnp.int32)
        tgt_a = wid * _CPW + lane
        tgt_b = tgt_a + 1

        zeros = jnp.zeros((16,), jnp.int32)
        full_n = jnp.full((16,), _N, jnp.int32)
        nm1 = jnp.full((16,), _N - 1, jnp.int32)

        def srch(_, carry):
            lo_a, hi_a, lo_b, hi_b = carry
            mid_a = (lo_a + hi_a) >> 1
            mid_b = (lo_b + hi_b) >> 1
            idx_v[pl.ds(0, 16)] = jnp.minimum(mid_a, nm1)
            idx_v[pl.ds(16, 16)] = jnp.minimum(mid_b, nm1)
            pltpu.async_copy(batch_hbm.at[idx_v], val_v, sem).wait()
            va = val_v[pl.ds(0, 16)]
            vb = val_v[pl.ds(16, 16)]
            lo_a2 = jnp.where(va < tgt_a, mid_a + 1, lo_a)
            hi_a2 = jnp.where(va < tgt_a, hi_a, mid_a)
            lo_b2 = jnp.where(vb < tgt_b, mid_b + 1, lo_b)
            hi_b2 = jnp.where(vb < tgt_b, hi_b, mid_b)
            lo_a3 = jnp.where(lo_a < hi_a, lo_a2, lo_a)
            hi_a3 = jnp.where(lo_a < hi_a, hi_a2, hi_a)
            lo_b3 = jnp.where(lo_b < hi_b, lo_b2, lo_b)
            hi_b3 = jnp.where(lo_b < hi_b, hi_b2, hi_b)
            return lo_a3, hi_a3, lo_b3, hi_b3

        starts, _, nexts, _ = lax.fori_loop(
            0, _SEARCH_ROUNDS, srch, (zeros, full_n, zeros, full_n))

        lens = jnp.minimum(jnp.int32(_W), nexts - starts)

        def one_case(j, acc):
            sel = jnp.full((16,), 0, jnp.int32) + j
            s_bc = starts.at[sel].get(mode="promise_in_bounds")
            l_bc = lens.at[sel].get(mode="promise_in_bounds")
            for k in range(_NVEC):
                widx[pl.ds(k * 16, 16)] = jnp.minimum(
                    s_bc + (k * 16) + lane, nm1)
            copies = []
            for (o, n) in _CHUNKS:
                copies.append(pltpu.make_async_copy(
                    pred_hbm.at[widx.at[pl.ds(o, n)]],
                    pbuf.at[pl.ds(o, n)], sem))
                copies.append(pltpu.make_async_copy(
                    true_hbm.at[widx.at[pl.ds(o, n)]],
                    tbuf.at[pl.ds(o, n)], sem))
            for c in copies:
                c.start()
            for c in copies:
                c.wait()
            for k in range(_NVEC):
                off = jnp.int32(k * 16) + lane
                p = pbuf[pl.ds(k * 16, 16)]
                t = tbuf[pl.ds(k * 16, 16)]
                d = p - t
                acc = acc + jnp.where(off < l_bc, d * d, jnp.float32(0.0))
            return acc

        acc = lax.fori_loop(0, _CPW, one_case,
                            jnp.zeros((16,), jnp.float32))
        acc_v[...] = acc
        pltpu.sync_copy(acc_v, out_hbm.at[wid])

    return sc_kernel


def kernel(batch, pred, true):
    partials = _sc_boundary_kernel()(batch.astype(jnp.int32), pred, true)
    s_all = _tc_sum_sq(pred, true)
    s_bnd = jnp.sum(partials)
    return (s_all + 9.0 * s_bnd) / _N


# speedup vs baseline: 2.8518x; 2.8518x over previous
import jax
import jax.numpy as jnp
from jax.experimental import pallas as pl
from jax.experimental.pallas import tpu as pltpu

_N = 8388608
_LANES = 128
_ROWS = _N // _LANES
_BLK_R = 16384
_GRID = _ROWS // _BLK_R
_CARRY_R = 6


def _body(b_ref, p_ref, t_ref, out_ref, carry_ref):
    k = pl.program_id(0)

    @pl.when(k == 0)
    def _init():
        carry_ref[...] = jnp.full((_CARRY_R, _LANES), -1, jnp.int32)
        out_ref[...] = jnp.zeros((1, 1), jnp.float32)

    b = b_ref[...]
    aug = jnp.concatenate([carry_ref[...], b], axis=0)
    shifted = jnp.concatenate(
        [aug[:_BLK_R, 96:], aug[1:_BLK_R + 1, :96]], axis=1)
    w = jnp.where(b != shifted, 10.0, 1.0).astype(jnp.float32)
    d = p_ref[...] - t_ref[...]
    s = jnp.sum(w * (d * d)).reshape(1, 1)
    out_ref[...] = out_ref[...] + s
    carry_ref[...] = b[_BLK_R - _CARRY_R:_BLK_R, :]


def kernel(batch, pred, true):
    b2 = batch.astype(jnp.int32).reshape(_ROWS, _LANES)
    p2 = pred.reshape(_ROWS, _LANES)
    t2 = true.reshape(_ROWS, _LANES)
    spec = pl.BlockSpec((_BLK_R, _LANES), lambda k: (k, 0))
    total = pl.pallas_call(
        _body,
        grid=(_GRID,),
        in_specs=[spec, spec, spec],
        out_specs=pl.BlockSpec((1, 1), lambda k: (0, 0)),
        out_shape=jax.ShapeDtypeStruct((1, 1), jnp.float32),
        scratch_shapes=[pltpu.VMEM((_CARRY_R, _LANES), jnp.int32)],
        compiler_params=pltpu.CompilerParams(
            vmem_limit_bytes=120 * 1024 * 1024),
    )(b2, p2, t2)
    return total[0, 0] / _N


# speedup vs baseline: 2.9904x; 1.0486x over previous
import jax
import jax.numpy as jnp
from jax.experimental import pallas as pl
from jax.experimental.pallas import tpu as pltpu

_N = 8388608
_LANES = 128
_ROWS = _N // _LANES
_BLK_R = 8192
_GRID = _ROWS // _BLK_R
_CARRY_R = 6


def _body(b_ref, p_ref, t_ref, out_ref, carry_ref):
    k = pl.program_id(0)

    @pl.when(k == 0)
    def _init():
        carry_ref[...] = jnp.full((_CARRY_R, _LANES), -1, jnp.int32)
        out_ref[...] = jnp.zeros((1, 1), jnp.float32)

    b = b_ref[...]
    aug = jnp.concatenate([carry_ref[...], b], axis=0)
    shifted = jnp.concatenate(
        [aug[:_BLK_R, 96:], aug[1:_BLK_R + 1, :96]], axis=1)
    w = jnp.where(b != shifted, 10.0, 1.0).astype(jnp.float32)
    d = p_ref[...] - t_ref[...]
    s = jnp.sum(w * (d * d)).reshape(1, 1)
    out_ref[...] = out_ref[...] + s
    carry_ref[...] = b[_BLK_R - _CARRY_R:_BLK_R, :]


def kernel(batch, pred, true):
    b2 = batch.astype(jnp.int32).reshape(_ROWS, _LANES)
    p2 = pred.reshape(_ROWS, _LANES)
    t2 = true.reshape(_ROWS, _LANES)
    spec = pl.BlockSpec((_BLK_R, _LANES), lambda k: (k, 0))
    total = pl.pallas_call(
        _body,
        grid=(_GRID,),
        in_specs=[spec, spec, spec],
        out_specs=pl.BlockSpec((1, 1), lambda k: (0, 0)),
        out_shape=jax.ShapeDtypeStruct((1, 1), jnp.float32),
        scratch_shapes=[pltpu.VMEM((_CARRY_R, _LANES), jnp.int32)],
        compiler_params=pltpu.CompilerParams(
            vmem_limit_bytes=120 * 1024 * 1024),
    )(b2, p2, t2)
    return total[0, 0] / _N


# speedup vs baseline: 3.0123x; 1.0073x over previous
import jax
import jax.numpy as jnp
from jax.experimental import pallas as pl
from jax.experimental.pallas import tpu as pltpu

_N = 8388608
_LANES = 128
_ROWS = _N // _LANES
_BLK_R = 8192
_GRID = _ROWS // _BLK_R
_CARRY_R = 6


def _body(b_ref, p_ref, t_ref, out_ref, carry_ref):
    k = pl.program_id(0)

    @pl.when(k == 0)
    def _init():
        carry_ref[...] = jnp.full((_CARRY_R, _LANES), -1, jnp.int32)
        out_ref[...] = jnp.zeros((1, 1), jnp.float32)

    b = b_ref[...]
    aug = jnp.concatenate([carry_ref[...], b], axis=0)
    shifted = jnp.concatenate(
        [aug[:_BLK_R, 96:], aug[1:_BLK_R + 1, :96]], axis=1)
    w = jnp.where(b != shifted, 10.0, 1.0).astype(jnp.float32)
    d = p_ref[...] - t_ref[...]
    s = jnp.sum(w * (d * d)).reshape(1, 1)
    out_ref[...] = out_ref[...] + s
    carry_ref[...] = b[_BLK_R - _CARRY_R:_BLK_R, :]


def kernel(batch, pred, true):
    b2 = batch.astype(jnp.int32).reshape(_ROWS, _LANES)
    p2 = pred.reshape(_ROWS, _LANES)
    t2 = true.reshape(_ROWS, _LANES)
    spec = pl.BlockSpec((_BLK_R, _LANES), lambda k: (k, 0))
    total = pl.pallas_call(
        _body,
        grid=(_GRID,),
        in_specs=[spec, spec, spec],
        out_specs=pl.BlockSpec((1, 1), lambda k: (0, 0)),
        out_shape=jax.ShapeDtypeStruct((1, 1), jnp.float32),
        scratch_shapes=[pltpu.VMEM((_CARRY_R, _LANES), jnp.int32)],
    )(b2, p2, t2)
    return total[0, 0] / _N


# speedup vs baseline: 3.1399x; 1.0424x over previous
import jax
import jax.numpy as jnp
from jax.experimental import pallas as pl
from jax.experimental.pallas import tpu as pltpu

_N = 8388608
_LANES = 128
_ROWS = _N // _LANES
_BLK_R = 8192
_GRID = _ROWS // _BLK_R
_CARRY_R = 6


def _body(b_ref, p_ref, t_ref, out_ref, carry_ref):
    k = pl.program_id(0)

    @pl.when(k == 0)
    def _init():
        carry_ref[...] = jnp.full((_CARRY_R, _LANES), -1, jnp.int32)
        out_ref[...] = jnp.zeros((1, 1), jnp.float32)

    b = b_ref[...]
    aug = jnp.concatenate([carry_ref[...], b], axis=0)
    rolled = pltpu.roll(aug, 32, 1)
    lane = jax.lax.broadcasted_iota(jnp.int32, (_BLK_R, _LANES), 1)
    shifted = jnp.where(lane >= 32, rolled[1:_BLK_R + 1, :],
                        rolled[:_BLK_R, :])
    w = jnp.where(b != shifted, 10.0, 1.0).astype(jnp.float32)
    d = p_ref[...] - t_ref[...]
    s = jnp.sum(w * (d * d)).reshape(1, 1)
    out_ref[...] = out_ref[...] + s
    carry_ref[...] = b[_BLK_R - _CARRY_R:_BLK_R, :]


def kernel(batch, pred, true):
    b2 = batch.astype(jnp.int32).reshape(_ROWS, _LANES)
    p2 = pred.reshape(_ROWS, _LANES)
    t2 = true.reshape(_ROWS, _LANES)
    spec = pl.BlockSpec((_BLK_R, _LANES), lambda k: (k, 0))
    total = pl.pallas_call(
        _body,
        grid=(_GRID,),
        in_specs=[spec, spec, spec],
        out_specs=pl.BlockSpec((1, 1), lambda k: (0, 0)),
        out_shape=jax.ShapeDtypeStruct((1, 1), jnp.float32),
        scratch_shapes=[pltpu.VMEM((_CARRY_R, _LANES), jnp.int32)],
    )(b2, p2, t2)
    return total[0, 0] / _N
